# baseline (speedup 1.0000x reference)
import functools

import jax
import jax.numpy as jnp
from jax import lax
from jax.experimental import pallas as pl
from jax.experimental.pallas import tpu as pltpu
from jax.experimental.pallas import tpu_sc as plsc


def _sc_colsum(x2, H, W):
    R, N = x2.shape
    info = plsc.get_sparse_core_info()
    NC, NS, L = info.num_cores, info.num_subcores, info.num_lanes
    NW = NC * NS
    per_w = R // NW
    NV = N // L

    mesh = plsc.VectorSubcoreMesh(core_axis_name="c", subcore_axis_name="s")

    @functools.partial(
        pl.kernel,
        mesh=mesh,
        out_type=jax.ShapeDtypeStruct((R, N), jnp.float32),
        scratch_types=[
            pltpu.VMEM((N,), jnp.float32),
            pltpu.VMEM((N,), jnp.float32),
        ],
    )
    def sc_kernel(x_hbm, out_hbm, img, cs):
        wid = lax.axis_index("s") * NC + lax.axis_index("c")

        for k in range(per_w):
            row = wid * per_w + k
            pltpu.sync_copy(x_hbm.at[row], img)

            def col_body(i, _):
                base = i * L
                r = base // W
                up = jnp.where(r == 0, base, base - W)
                dn = jnp.where(r == H - 1, base, base + W)
                cs[pl.ds(base, L)] = (
                    img[pl.ds(up, L)]
                    + img[pl.ds(base, L)]
                    + img[pl.ds(dn, L)]
                )
                return 0

            lax.fori_loop(0, NV, col_body, 0)
            pltpu.sync_copy(cs, out_hbm.at[row])

    return sc_kernel(x2)


def _tc_kernel(x_ref, cs_ref, w_ref, b_ref, out_ref, *, W):
    xb = x_ref[0]
    colsum = cs_ref[0]
    NB = xb.shape[1]

    wpos = jax.lax.broadcasted_iota(jnp.int32, (1, NB), 1) % W
    left = jnp.concatenate([colsum[:, :1], colsum[:, :-1]], axis=1)
    left = jnp.where(wpos == 0, colsum, left)
    right = jnp.concatenate([colsum[:, 1:], colsum[:, -1:]], axis=1)
    right = jnp.where(wpos == W - 1, colsum, right)
    mean = (left + colsum + right - xb) * 0.125

    agg = jnp.concatenate([xb, mean], axis=0)
    y = jnp.dot(w_ref[...], agg, preferred_element_type=jnp.float32)
    out_ref[0] = jnp.maximum(y + b_ref[...], 0.0) + xb


def kernel(x, W_proj, b_proj):
    B, C, H, W = x.shape
    N = H * W
    HB = 28
    nbands = H // HB
    NB = HB * W

    xr = x.reshape(B * C, N)
    colsum = _sc_colsum(xr, H, W).reshape(B, C, N)
    x2 = x.reshape(B, C, N)
    b2 = b_proj.reshape(C, 1)

    grid = (B, nbands)
    out = pl.pallas_call(
        functools.partial(_tc_kernel, W=W),
        grid=grid,
        in_specs=[
            pl.BlockSpec((1, C, NB), lambda b, h: (b, 0, h)),
            pl.BlockSpec((1, C, NB), lambda b, h: (b, 0, h)),
            pl.BlockSpec((C, 2 * C), lambda b, h: (0, 0)),
            pl.BlockSpec((C, 1), lambda b, h: (0, 0)),
        ],
        out_specs=pl.BlockSpec((1, C, NB), lambda b, h: (b, 0, h)),
        out_shape=jax.ShapeDtypeStruct((B, C, N), jnp.float32),
        compiler_params=pltpu.CompilerParams(
            dimension_semantics=("parallel", "arbitrary"),
        ),
    )(x2, colsum, W_proj, b2)
    return out.reshape(B, C, H, W)


# speedup vs baseline: 2.5786x; 2.5786x over previous
import functools

import jax
import jax.numpy as jnp
from jax.experimental import pallas as pl
from jax.experimental.pallas import tpu as pltpu


def _band_kernel(cur_ref, up_ref, down_ref, w_ref, b_ref, out_ref, *, W, HB):
    NB = HB * W
    h = pl.program_id(1)
    nbands = pl.num_programs(1)
    cur = cur_ref[0]

    up_blk = up_ref[0]
    down_blk = down_ref[0]
    up_row = jnp.where(h == 0, up_blk[:, :W], up_blk[:, 3 * W:])
    down_row = jnp.where(h == nbands - 1, down_blk[:, 3 * W:], down_blk[:, :W])

    up = jnp.concatenate([up_row, cur[:, : NB - W]], axis=1)
    down = jnp.concatenate([cur[:, W:], down_row], axis=1)
    colsum = up + cur + down

    wpos = jax.lax.broadcasted_iota(jnp.int32, (1, NB), 1) % W
    left = jnp.concatenate([colsum[:, :1], colsum[:, :-1]], axis=1)
    left = jnp.where(wpos == 0, colsum, left)
    right = jnp.concatenate([colsum[:, 1:], colsum[:, -1:]], axis=1)
    right = jnp.where(wpos == W - 1, colsum, right)
    sum9 = left + colsum + right

    mean = (sum9 - cur) * 0.125

    agg = jnp.concatenate([cur, mean], axis=0)
    y = jnp.dot(w_ref[...], agg, preferred_element_type=jnp.float32)
    y = y + b_ref[...]
    out_ref[0] = jnp.maximum(y, 0.0) + cur


def kernel(x, W_proj, b_proj):
    B, C, H, W = x.shape
    N = H * W
    HB = 28
    nbands = H // HB
    NB = HB * W
    RPB = HB // 4

    x2 = x.reshape(B, C, N)
    b2 = b_proj.reshape(C, 1)

    grid = (B, nbands)
    out = pl.pallas_call(
        functools.partial(_band_kernel, W=W, HB=HB),
        grid=grid,
        in_specs=[
            pl.BlockSpec((1, C, NB), lambda b, h: (b, 0, h)),
            pl.BlockSpec(
                (1, C, 4 * W),
                lambda b, h: (b, 0, jnp.maximum(h * RPB - 1, 0)),
            ),
            pl.BlockSpec(
                (1, C, 4 * W),
                lambda b, h: (b, 0, jnp.minimum((h + 1) * RPB, nbands * RPB - 1)),
            ),
            pl.BlockSpec((C, 2 * C), lambda b, h: (0, 0)),
            pl.BlockSpec((C, 1), lambda b, h: (0, 0)),
        ],
        out_specs=pl.BlockSpec((1, C, NB), lambda b, h: (b, 0, h)),
        out_shape=jax.ShapeDtypeStruct((B, C, N), jnp.float32),
        compiler_params=pltpu.CompilerParams(
            dimension_semantics=("parallel", "arbitrary"),
        ),
    )(x2, x2, x2, W_proj, b2)
    return out.reshape(B, C, H, W)
